# Initial kernel scaffold; baseline (speedup 1.0000x reference)
import jax
import jax.numpy as jnp
from jax.experimental import pallas as pl


def kernel(nodes, edges, senders, receivers, globals_, We1, be1, We2, be2, Wn1, bn1, Wn2, bn2, Wg1, bg1, Wg2, bg2):
    raise NotImplementedError("write your pallas kernel here")


# speedup vs baseline: 4.1796x; 4.1796x over previous
import functools

import jax
import jax.numpy as jnp
from jax import lax
from jax.experimental import pallas as pl
from jax.experimental.pallas import tpu as pltpu
from jax.experimental.pallas import tpu_sc as plsc

F32 = jnp.float32

_N = 10000
_E = 320000
_DF = 128
_DE = 16
_DG = 16
_L = 16

_NC = 2
_NS = 16
_NW = _NC * _NS
_EPW = _E // _NW
_TB = 125
_ROWS_PER_W = _EPW // _TB
_CH = 1000
_TPC = _CH // _TB
_NCHUNK = _EPW // _CH
_NPS = _N // _NS

def _sc_gather_body(ps_hbm, pr_hbm, snd_hbm, rcv_hbm, gs_hbm, gr_hbm,
                    idx_s, idx_r, rows_s, rows_r, sem):
    cid = lax.axis_index("c")
    sid = lax.axis_index("s")
    wid = cid * _NS + sid
    base_row = wid * _ROWS_PER_W
    base_e = wid * _EPW

    pltpu.sync_copy(snd_hbm.at[pl.ds(base_row, _ROWS_PER_W)], idx_s)
    pltpu.sync_copy(rcv_hbm.at[pl.ds(base_row, _ROWS_PER_W)], idx_r)

    def chunk(t, carry):
        descs = []
        for j in range(_TPC):
            r = t * _TPC + j
            dst = pl.ds(j * _TB, _TB)
            descs.append(pltpu.async_copy(ps_hbm.at[idx_s.at[r]],
                                          rows_s.at[dst], sem))
            descs.append(pltpu.async_copy(pr_hbm.at[idx_r.at[r]],
                                          rows_r.at[dst], sem))
        for d in descs:
            d.wait()
        pltpu.sync_copy(rows_s, gs_hbm.at[pl.ds(base_e + t * _CH, _CH)])
        pltpu.sync_copy(rows_r, gr_hbm.at[pl.ds(base_e + t * _CH, _CH)])
        return carry

    lax.fori_loop(0, _NCHUNK, chunk, 0)


def _sc_gather(ps, pr, snd2d, rcv2d):
    mesh = plsc.VectorSubcoreMesh(core_axis_name="c", subcore_axis_name="s")
    k = functools.partial(
        pl.kernel,
        out_type=(
            jax.ShapeDtypeStruct((_E, _L), F32),
            jax.ShapeDtypeStruct((_E, _L), F32),
        ),
        mesh=mesh,
        scratch_types=[
            pltpu.VMEM((_ROWS_PER_W, _TB), jnp.int32),
            pltpu.VMEM((_ROWS_PER_W, _TB), jnp.int32),
            pltpu.VMEM((_CH, _L), F32),
            pltpu.VMEM((_CH, _L), F32),
            pltpu.SemaphoreType.DMA,
        ],
        compiler_params=pltpu.CompilerParams(use_tc_tiling_on_sc=False),
    )(_sc_gather_body)
    return k(ps, pr, snd2d, rcv2d)


def _sc_scatter_body(ne_hbm, snd_hbm, rcv_hbm, zeros_hbm, out_hbm,
                     idx_s, idx_r, rows, sh_s, sh_r):
    cid = lax.axis_index("c")
    sid = lax.axis_index("s")
    wid = cid * _NS + sid
    zslice = pl.ds(sid * _NPS, _NPS)

    pltpu.sync_copy(zeros_hbm.at[zslice], sh_s.at[zslice])
    pltpu.sync_copy(zeros_hbm.at[zslice], sh_r.at[zslice])
    pltpu.sync_copy(snd_hbm.at[pl.ds(wid * _ROWS_PER_W, _ROWS_PER_W)], idx_s)
    pltpu.sync_copy(rcv_hbm.at[pl.ds(wid * _ROWS_PER_W, _ROWS_PER_W)], idx_r)
    plsc.subcore_barrier()

    def chunk(t, carry):
        pltpu.sync_copy(ne_hbm.at[pl.ds(wid * _EPW + t * _CH, _CH)], rows)
        for j in range(_TPC):
            r = t * _TPC + j
            src = rows.at[pl.ds(j * _TB, _TB)]
            pltpu.sync_copy(src, sh_s.at[idx_s.at[r]], add=True)
            pltpu.sync_copy(src, sh_r.at[idx_r.at[r]], add=True)
        return carry

    lax.fori_loop(0, _NCHUNK, chunk, 0)
    plsc.subcore_barrier()

    pltpu.sync_copy(sh_s.at[zslice], out_hbm.at[cid, 0, zslice])
    pltpu.sync_copy(sh_r.at[zslice], out_hbm.at[cid, 1, zslice])


def _sc_scatter(ne, snd2d, rcv2d, zeros):
    mesh = plsc.VectorSubcoreMesh(core_axis_name="c", subcore_axis_name="s")
    k = functools.partial(
        pl.kernel,
        out_type=jax.ShapeDtypeStruct((_NC, 2, _N, _L), F32),
        mesh=mesh,
        scratch_types=[
            pltpu.VMEM((_ROWS_PER_W, _TB), jnp.int32),
            pltpu.VMEM((_ROWS_PER_W, _TB), jnp.int32),
            pltpu.VMEM((_CH, _L), F32),
            pltpu.VMEM_SHARED((_N, _L), F32),
            pltpu.VMEM_SHARED((_N, _L), F32),
        ],
        compiler_params=pltpu.CompilerParams(use_tc_tiling_on_sc=False),
    )(_sc_scatter_body)
    return k(ne, snd2d, rcv2d, zeros)


def _tc_project_body(n_ref, ws_ref, wr_ref, ps_ref, pr_ref):
    n = n_ref[...]
    ps_ref[...] = jnp.dot(n, ws_ref[...], preferred_element_type=F32)
    pr_ref[...] = jnp.dot(n, wr_ref[...], preferred_element_type=F32)


def _tc_project(nodes, ws, wr):
    bn = 2000
    grid = _N // bn
    return pl.pallas_call(
        _tc_project_body,
        grid=(grid,),
        in_specs=[
            pl.BlockSpec((bn, _DF), lambda i: (i, 0)),
            pl.BlockSpec((_DF, _L), lambda i: (0, 0)),
            pl.BlockSpec((_DF, _L), lambda i: (0, 0)),
        ],
        out_specs=[
            pl.BlockSpec((bn, _L), lambda i: (i, 0)),
            pl.BlockSpec((bn, _L), lambda i: (i, 0)),
        ],
        out_shape=[
            jax.ShapeDtypeStruct((_N, _L), F32),
            jax.ShapeDtypeStruct((_N, _L), F32),
        ],
    )(nodes, ws, wr)


def _tc_edge_body(e_ref, gs_ref, gr_ref, we1e_ref, we1g_ref, be1_ref,
                  we2_ref, be2_ref, g_ref, ne_ref, esum_ref):
    cst = (jnp.dot(g_ref[...], we1g_ref[...], preferred_element_type=F32)
           + be1_ref[...])
    pre = (jnp.dot(e_ref[...], we1e_ref[...], preferred_element_type=F32)
           + gs_ref[...] + gr_ref[...] + cst)
    h = jnp.maximum(pre, 0.0)
    ne = jnp.dot(h, we2_ref[...], preferred_element_type=F32) + be2_ref[...]
    ne_ref[...] = ne
    esum_ref[0, 0, :] = jnp.sum(ne, axis=0)


def _tc_edge(edges, gs, gr, we1e, we1g, be1, we2, be2, g):
    be = 4000
    grid = _E // be
    return pl.pallas_call(
        _tc_edge_body,
        grid=(grid,),
        in_specs=[
            pl.BlockSpec((be, _DE), lambda i: (i, 0)),
            pl.BlockSpec((be, _L), lambda i: (i, 0)),
            pl.BlockSpec((be, _L), lambda i: (i, 0)),
            pl.BlockSpec((_DE, _L), lambda i: (0, 0)),
            pl.BlockSpec((_DG, _L), lambda i: (0, 0)),
            pl.BlockSpec((1, _L), lambda i: (0, 0)),
            pl.BlockSpec((_L, _L), lambda i: (0, 0)),
            pl.BlockSpec((1, _L), lambda i: (0, 0)),
            pl.BlockSpec((1, _DG), lambda i: (0, 0)),
        ],
        out_specs=[
            pl.BlockSpec((be, _L), lambda i: (i, 0)),
            pl.BlockSpec((1, 1, _L), lambda i: (i, 0, 0)),
        ],
        out_shape=[
            jax.ShapeDtypeStruct((_E, _L), F32),
            jax.ShapeDtypeStruct((grid, 1, _L), F32),
        ],
    )(edges, gs, gr, we1e, we1g, be1, we2, be2, g)


def _tc_node_body(n_ref, p00_ref, p01_ref, p10_ref, p11_ref, esum_ref,
                  wn1x_ref, wn1s_ref, wn1r_ref, wn1g_ref, bn1_ref,
                  wn2_ref, bn2_ref, g_ref,
                  wg1n_ref, wg1e_ref, wg1g_ref, bg1_ref, wg2_ref, bg2_ref,
                  nn_ref, ng_ref, nsum_ref):
    sagg = p00_ref[...] + p10_ref[...]
    ragg = p01_ref[...] + p11_ref[...]
    cst = (jnp.dot(g_ref[...], wn1g_ref[...], preferred_element_type=F32)
           + bn1_ref[...])
    pre = (jnp.dot(n_ref[...], wn1x_ref[...], preferred_element_type=F32)
           + jnp.dot(sagg, wn1s_ref[...], preferred_element_type=F32)
           + jnp.dot(ragg, wn1r_ref[...], preferred_element_type=F32)
           + cst)
    nn = (jnp.dot(jnp.maximum(pre, 0.0), wn2_ref[...],
                  preferred_element_type=F32) + bn2_ref[...])
    nn_ref[...] = nn

    i = pl.program_id(0)
    s = jnp.sum(nn, axis=0, keepdims=True)

    @pl.when(i == 0)
    def _():
        nsum_ref[...] = s

    @pl.when(i > 0)
    def _():
        nsum_ref[...] = nsum_ref[...] + s

    @pl.when(i == pl.num_programs(0) - 1)
    def _():
        node_mean = nsum_ref[...] * (1.0 / _N)
        edge_mean = (jnp.sum(esum_ref[...], axis=(0, 1), keepdims=False)
                     * (1.0 / _E)).reshape(1, _L)
        gpre = (jnp.dot(node_mean, wg1n_ref[...], preferred_element_type=F32)
                + jnp.dot(edge_mean, wg1e_ref[...], preferred_element_type=F32)
                + jnp.dot(g_ref[...], wg1g_ref[...], preferred_element_type=F32)
                + bg1_ref[...])
        ng_ref[...] = (jnp.dot(jnp.maximum(gpre, 0.0), wg2_ref[...],
                               preferred_element_type=F32) + bg2_ref[...])


def _tc_node(nodes, p00, p01, p10, p11, esums,
             wn1x, wn1s, wn1r, wn1g, bn1, wn2, bn2, g,
             wg1n, wg1e, wg1g, bg1, wg2, bg2):
    bn = 2000
    grid = _N // bn
    nblk = esums.shape[0]
    w16 = lambda i: (0, 0)
    return pl.pallas_call(
        _tc_node_body,
        grid=(grid,),
        in_specs=[
            pl.BlockSpec((bn, _DF), lambda i: (i, 0)),
            pl.BlockSpec((bn, _L), lambda i: (i, 0)),
            pl.BlockSpec((bn, _L), lambda i: (i, 0)),
            pl.BlockSpec((bn, _L), lambda i: (i, 0)),
            pl.BlockSpec((bn, _L), lambda i: (i, 0)),
            pl.BlockSpec((nblk, 1, _L), lambda i: (0, 0, 0)),
            pl.BlockSpec((_DF, _L), w16),
            pl.BlockSpec((_L, _L), w16),
            pl.BlockSpec((_L, _L), w16),
            pl.BlockSpec((_DG, _L), w16),
            pl.BlockSpec((1, _L), w16),
            pl.BlockSpec((_L, _L), w16),
            pl.BlockSpec((1, _L), w16),
            pl.BlockSpec((1, _DG), w16),
            pl.BlockSpec((_L, _L), w16),
            pl.BlockSpec((_L, _L), w16),
            pl.BlockSpec((_DG, _L), w16),
            pl.BlockSpec((1, _L), w16),
            pl.BlockSpec((_L, _L), w16),
            pl.BlockSpec((1, _L), w16),
        ],
        out_specs=[
            pl.BlockSpec((bn, _L), lambda i: (i, 0)),
            pl.BlockSpec((1, _L), lambda i: (0, 0)),
        ],
        out_shape=[
            jax.ShapeDtypeStruct((_N, _L), F32),
            jax.ShapeDtypeStruct((1, _L), F32),
        ],
        scratch_shapes=[pltpu.VMEM((1, _L), F32)],
    )(nodes, p00, p01, p10, p11, esums,
      wn1x, wn1s, wn1r, wn1g, bn1, wn2, bn2, g,
      wg1n, wg1e, wg1g, bg1, wg2, bg2)


def kernel(nodes, edges, senders, receivers, globals_,
           We1, be1, We2, be2, Wn1, bn1, Wn2, bn2, Wg1, bg1, Wg2, bg2):
    we1e = We1[:_DE]
    we1s = We1[_DE:_DE + _DF]
    we1r = We1[_DE + _DF:_DE + 2 * _DF]
    we1g = We1[_DE + 2 * _DF:]
    wn1x = Wn1[:_DF]
    wn1s = Wn1[_DF:_DF + _L]
    wn1r = Wn1[_DF + _L:_DF + 2 * _L]
    wn1g = Wn1[_DF + 2 * _L:]
    wg1n = Wg1[:_L]
    wg1e = Wg1[_L:2 * _L]
    wg1g = Wg1[2 * _L:]
    be1r = be1.reshape(1, _L)
    be2r = be2.reshape(1, _L)
    bn1r = bn1.reshape(1, _L)
    bn2r = bn2.reshape(1, _L)
    bg1r = bg1.reshape(1, _L)
    bg2r = bg2.reshape(1, _L)
    snd2d = senders.reshape(_E // _TB, _TB)
    rcv2d = receivers.reshape(_E // _TB, _TB)
    zeros = jnp.zeros((_N, _L), F32)

    ps, pr = _tc_project(nodes, we1s, we1r)
    gs, gr = _sc_gather(ps, pr, snd2d, rcv2d)
    new_edges, esums = _tc_edge(edges, gs, gr, we1e, we1g, be1r,
                                We2, be2r, globals_)
    partials = _sc_scatter(new_edges, snd2d, rcv2d, zeros)
    new_nodes, new_globals = _tc_node(
        nodes, partials[0, 0], partials[0, 1], partials[1, 0], partials[1, 1],
        esums, wn1x, wn1s, wn1r, wn1g, bn1r, Wn2, bn2r, globals_,
        wg1n, wg1e, wg1g, bg1r, Wg2, bg2r)
    return new_nodes, new_edges, new_globals


# speedup vs baseline: 8.7007x; 2.0817x over previous
import functools

import jax
import jax.numpy as jnp
from jax import lax
from jax.experimental import pallas as pl
from jax.experimental.pallas import tpu as pltpu
from jax.experimental.pallas import tpu_sc as plsc

F32 = jnp.float32

_N = 10000
_E = 320000
_DF = 128
_DE = 16
_DG = 16
_L = 16

_NC = 2
_NS = 16
_NW = _NC * _NS
_EPW = _E // _NW
_TB = 125
_ROWS_PER_W = _EPW // _TB
_CH = 1000
_TPC = _CH // _TB
_NCHUNK = _EPW // _CH
_NPS = _N // _NS

def _sc_gather_body(ps_hbm, pr_hbm, snd_hbm, rcv_hbm, gs_hbm, gr_hbm,
                    idx_s, idx_r, rows_s, rows_r, sem):
    cid = lax.axis_index("c")
    sid = lax.axis_index("s")
    wid = cid * _NS + sid
    base_row = wid * _ROWS_PER_W
    base_e = wid * _EPW

    pltpu.sync_copy(snd_hbm.at[pl.ds(base_row, _ROWS_PER_W)], idx_s)
    pltpu.sync_copy(rcv_hbm.at[pl.ds(base_row, _ROWS_PER_W)], idx_r)

    def chunk(t, carry):
        descs = []
        for j in range(_TPC):
            r = t * _TPC + j
            dst = pl.ds(j * _TB, _TB)
            descs.append(pltpu.async_copy(ps_hbm.at[idx_s.at[r]],
                                          rows_s.at[dst], sem))
            descs.append(pltpu.async_copy(pr_hbm.at[idx_r.at[r]],
                                          rows_r.at[dst], sem))
        for d in descs:
            d.wait()
        pltpu.sync_copy(rows_s, gs_hbm.at[pl.ds(base_e + t * _CH, _CH)])
        pltpu.sync_copy(rows_r, gr_hbm.at[pl.ds(base_e + t * _CH, _CH)])
        return carry

    lax.fori_loop(0, _NCHUNK, chunk, 0)


def _sc_gather(ps, pr, snd2d, rcv2d):
    mesh = plsc.VectorSubcoreMesh(core_axis_name="c", subcore_axis_name="s")
    k = functools.partial(
        pl.kernel,
        out_type=(
            jax.ShapeDtypeStruct((_E, _L), F32),
            jax.ShapeDtypeStruct((_E, _L), F32),
        ),
        mesh=mesh,
        scratch_types=[
            pltpu.VMEM((_ROWS_PER_W, _TB), jnp.int32),
            pltpu.VMEM((_ROWS_PER_W, _TB), jnp.int32),
            pltpu.VMEM((_CH, _L), F32),
            pltpu.VMEM((_CH, _L), F32),
            pltpu.SemaphoreType.DMA,
        ],
        compiler_params=pltpu.CompilerParams(use_tc_tiling_on_sc=False),
    )(_sc_gather_body)
    return k(ps, pr, snd2d, rcv2d)


def _sc_scatter_body(ne_hbm, snd_hbm, rcv_hbm, zeros_hbm, out_hbm,
                     idx_s, idx_r, rows, sh_s, sh_r):
    cid = lax.axis_index("c")
    sid = lax.axis_index("s")
    wid = cid * _NS + sid
    zslice = pl.ds(sid * _NPS, _NPS)

    pltpu.sync_copy(zeros_hbm.at[zslice], sh_s.at[zslice])
    pltpu.sync_copy(zeros_hbm.at[zslice], sh_r.at[zslice])
    pltpu.sync_copy(snd_hbm.at[pl.ds(wid * _ROWS_PER_W, _ROWS_PER_W)], idx_s)
    pltpu.sync_copy(rcv_hbm.at[pl.ds(wid * _ROWS_PER_W, _ROWS_PER_W)], idx_r)
    plsc.subcore_barrier()

    def chunk(t, carry):
        pltpu.sync_copy(ne_hbm.at[pl.ds(wid * _EPW + t * _CH, _CH)], rows)
        for j in range(_TPC):
            r = t * _TPC + j
            src = rows.at[pl.ds(j * _TB, _TB)]
            pltpu.sync_copy(src, sh_s.at[idx_s.at[r]], add=True)
            pltpu.sync_copy(src, sh_r.at[idx_r.at[r]], add=True)
        return carry

    lax.fori_loop(0, _NCHUNK, chunk, 0)
    plsc.subcore_barrier()

    pltpu.sync_copy(sh_s.at[zslice], out_hbm.at[cid, 0, zslice])
    pltpu.sync_copy(sh_r.at[zslice], out_hbm.at[cid, 1, zslice])


def _sc_scatter(ne, snd2d, rcv2d, zeros):
    mesh = plsc.VectorSubcoreMesh(core_axis_name="c", subcore_axis_name="s")
    k = functools.partial(
        pl.kernel,
        out_type=jax.ShapeDtypeStruct((_NC, 2, _N, _L), F32),
        mesh=mesh,
        scratch_types=[
            pltpu.VMEM((_ROWS_PER_W, _TB), jnp.int32),
            pltpu.VMEM((_ROWS_PER_W, _TB), jnp.int32),
            pltpu.VMEM((_CH, _L), F32),
            pltpu.VMEM_SHARED((_N, _L), F32),
            pltpu.VMEM_SHARED((_N, _L), F32),
        ],
        compiler_params=pltpu.CompilerParams(use_tc_tiling_on_sc=False),
    )(_sc_scatter_body)
    return k(ne, snd2d, rcv2d, zeros)


def _tc_project_body(n_ref, ws_ref, wr_ref, ps_ref, pr_ref):
    n = n_ref[...]
    ps_ref[...] = jnp.dot(n, ws_ref[...], preferred_element_type=F32)
    pr_ref[...] = jnp.dot(n, wr_ref[...], preferred_element_type=F32)


def _tc_project(nodes, ws, wr):
    bn = 2000
    grid = _N // bn
    return pl.pallas_call(
        _tc_project_body,
        grid=(grid,),
        in_specs=[
            pl.BlockSpec((bn, _DF), lambda i: (i, 0)),
            pl.BlockSpec((_DF, _L), lambda i: (0, 0)),
            pl.BlockSpec((_DF, _L), lambda i: (0, 0)),
        ],
        out_specs=[
            pl.BlockSpec((bn, _L), lambda i: (i, 0)),
            pl.BlockSpec((bn, _L), lambda i: (i, 0)),
        ],
        out_shape=[
            jax.ShapeDtypeStruct((_N, _L), F32),
            jax.ShapeDtypeStruct((_N, _L), F32),
        ],
    )(nodes, ws, wr)


_EP = _E // 8


def _tc_edge_body(e_ref, gs_ref, gr_ref, web1_ref, we1gt_ref, be1t_ref,
                  web2_ref, be2t_ref, g_ref, ne_ref, esum_ref):
    cst = (jnp.dot(g_ref[...], we1gt_ref[...], preferred_element_type=F32)
           + be1t_ref[...])
    pre = (jnp.dot(e_ref[...], web1_ref[...], preferred_element_type=F32)
           + gs_ref[...] + gr_ref[...] + cst)
    h = jnp.maximum(pre, 0.0)
    ne = jnp.dot(h, web2_ref[...], preferred_element_type=F32) + be2t_ref[...]
    ne_ref[...] = ne
    esum_ref[0, 0, :] = jnp.sum(ne, axis=0)


def _tc_edge(edges_p, gs_p, gr_p, web1, we1gt, be1t, web2, be2t, g):
    be = 4000
    grid = _EP // be
    return pl.pallas_call(
        _tc_edge_body,
        grid=(grid,),
        in_specs=[
            pl.BlockSpec((be, 128), lambda i: (i, 0)),
            pl.BlockSpec((be, 128), lambda i: (i, 0)),
            pl.BlockSpec((be, 128), lambda i: (i, 0)),
            pl.BlockSpec((128, 128), lambda i: (0, 0)),
            pl.BlockSpec((_DG, 128), lambda i: (0, 0)),
            pl.BlockSpec((1, 128), lambda i: (0, 0)),
            pl.BlockSpec((128, 128), lambda i: (0, 0)),
            pl.BlockSpec((1, 128), lambda i: (0, 0)),
            pl.BlockSpec((1, _DG), lambda i: (0, 0)),
        ],
        out_specs=[
            pl.BlockSpec((be, 128), lambda i: (i, 0)),
            pl.BlockSpec((1, 1, 128), lambda i: (i, 0, 0)),
        ],
        out_shape=[
            jax.ShapeDtypeStruct((_EP, 128), F32),
            jax.ShapeDtypeStruct((grid, 1, 128), F32),
        ],
    )(edges_p, gs_p, gr_p, web1, we1gt, be1t, web2, be2t, g)


def _tc_node_body(n_ref, p00_ref, p01_ref, p10_ref, p11_ref, esum_ref,
                  wn1x_ref, wn1s_ref, wn1r_ref, wn1g_ref, bn1_ref,
                  wn2_ref, bn2_ref, g_ref,
                  wg1n_ref, wg1e_ref, wg1g_ref, bg1_ref, wg2_ref, bg2_ref,
                  nn_ref, ng_ref, nsum_ref):
    sagg = p00_ref[...] + p10_ref[...]
    ragg = p01_ref[...] + p11_ref[...]
    cst = (jnp.dot(g_ref[...], wn1g_ref[...], preferred_element_type=F32)
           + bn1_ref[...])
    pre = (jnp.dot(n_ref[...], wn1x_ref[...], preferred_element_type=F32)
           + jnp.dot(sagg, wn1s_ref[...], preferred_element_type=F32)
           + jnp.dot(ragg, wn1r_ref[...], preferred_element_type=F32)
           + cst)
    nn = (jnp.dot(jnp.maximum(pre, 0.0), wn2_ref[...],
                  preferred_element_type=F32) + bn2_ref[...])
    nn_ref[...] = nn

    i = pl.program_id(0)
    s = jnp.sum(nn, axis=0, keepdims=True)

    @pl.when(i == 0)
    def _():
        nsum_ref[...] = s

    @pl.when(i > 0)
    def _():
        nsum_ref[...] = nsum_ref[...] + s

    @pl.when(i == pl.num_programs(0) - 1)
    def _():
        node_mean = nsum_ref[...] * (1.0 / _N)
        etot = jnp.sum(esum_ref[...], axis=(0, 1)).reshape(1, 128)
        gpre = (jnp.dot(node_mean, wg1n_ref[...], preferred_element_type=F32)
                + jnp.dot(etot, wg1e_ref[...],
                          preferred_element_type=F32) * (1.0 / _E)
                + jnp.dot(g_ref[...], wg1g_ref[...], preferred_element_type=F32)
                + bg1_ref[...])
        ng_ref[...] = (jnp.dot(jnp.maximum(gpre, 0.0), wg2_ref[...],
                               preferred_element_type=F32) + bg2_ref[...])


def _tc_node(nodes, p00, p01, p10, p11, esums,
             wn1x, wn1s, wn1r, wn1g, bn1, wn2, bn2, g,
             wg1n, wg1e, wg1g, bg1, wg2, bg2):
    bn = 2000
    grid = _N // bn
    nblk = esums.shape[0]
    w16 = lambda i: (0, 0)
    return pl.pallas_call(
        _tc_node_body,
        grid=(grid,),
        in_specs=[
            pl.BlockSpec((bn, _DF), lambda i: (i, 0)),
            pl.BlockSpec((bn, _L), lambda i: (i, 0)),
            pl.BlockSpec((bn, _L), lambda i: (i, 0)),
            pl.BlockSpec((bn, _L), lambda i: (i, 0)),
            pl.BlockSpec((bn, _L), lambda i: (i, 0)),
            pl.BlockSpec((nblk, 1, 128), lambda i: (0, 0, 0)),
            pl.BlockSpec((_DF, _L), w16),
            pl.BlockSpec((_L, _L), w16),
            pl.BlockSpec((_L, _L), w16),
            pl.BlockSpec((_DG, _L), w16),
            pl.BlockSpec((1, _L), w16),
            pl.BlockSpec((_L, _L), w16),
            pl.BlockSpec((1, _L), w16),
            pl.BlockSpec((1, _DG), w16),
            pl.BlockSpec((_L, _L), w16),
            pl.BlockSpec((128, _L), w16),
            pl.BlockSpec((_DG, _L), w16),
            pl.BlockSpec((1, _L), w16),
            pl.BlockSpec((_L, _L), w16),
            pl.BlockSpec((1, _L), w16),
        ],
        out_specs=[
            pl.BlockSpec((bn, _L), lambda i: (i, 0)),
            pl.BlockSpec((1, _L), lambda i: (0, 0)),
        ],
        out_shape=[
            jax.ShapeDtypeStruct((_N, _L), F32),
            jax.ShapeDtypeStruct((1, _L), F32),
        ],
        scratch_shapes=[pltpu.VMEM((1, _L), F32)],
    )(nodes, p00, p01, p10, p11, esums,
      wn1x, wn1s, wn1r, wn1g, bn1, wn2, bn2, g,
      wg1n, wg1e, wg1g, bg1, wg2, bg2)


def kernel(nodes, edges, senders, receivers, globals_,
           We1, be1, We2, be2, Wn1, bn1, Wn2, bn2, Wg1, bg1, Wg2, bg2):
    we1e = We1[:_DE]
    we1s = We1[_DE:_DE + _DF]
    we1r = We1[_DE + _DF:_DE + 2 * _DF]
    we1g = We1[_DE + 2 * _DF:]
    wn1x = Wn1[:_DF]
    wn1s = Wn1[_DF:_DF + _L]
    wn1r = Wn1[_DF + _L:_DF + 2 * _L]
    wn1g = Wn1[_DF + 2 * _L:]
    wg1n = Wg1[:_L]
    wg1e = Wg1[_L:2 * _L]
    wg1g = Wg1[2 * _L:]
    be1r = be1.reshape(1, _L)
    be2r = be2.reshape(1, _L)
    bn1r = bn1.reshape(1, _L)
    bn2r = bn2.reshape(1, _L)
    bg1r = bg1.reshape(1, _L)
    bg2r = bg2.reshape(1, _L)
    snd2d = senders.reshape(_E // _TB, _TB)
    rcv2d = receivers.reshape(_E // _TB, _TB)
    zeros = jnp.zeros((_N, _L), F32)

    eye8 = jnp.eye(8, dtype=F32)
    web1 = jnp.kron(eye8, we1e)
    web2 = jnp.kron(eye8, We2)
    we1gt = jnp.tile(we1g, (1, 8))
    be1t = jnp.tile(be1r, (1, 8))
    be2t = jnp.tile(be2r, (1, 8))
    wg1et = jnp.tile(wg1e, (8, 1))

    edges_p = edges.reshape(_EP, 128)

    ps, pr = _tc_project(nodes, we1s, we1r)
    gs, gr = _sc_gather(ps, pr, snd2d, rcv2d)
    ne_p, esums = _tc_edge(edges_p, gs.reshape(_EP, 128), gr.reshape(_EP, 128),
                           web1, we1gt, be1t, web2, be2t, globals_)
    new_edges = ne_p.reshape(_E, _L)
    partials = _sc_scatter(new_edges, snd2d, rcv2d, zeros)
    new_nodes, new_globals = _tc_node(
        nodes, partials[0, 0], partials[0, 1], partials[1, 0], partials[1, 1],
        esums, wn1x, wn1s, wn1r, wn1g, bn1r, Wn2, bn2r, globals_,
        wg1n, wg1et, wg1g, bg1r, Wg2, bg2r)
    return new_nodes, new_edges, new_globals


# speedup vs baseline: 9.2400x; 1.0620x over previous
import functools

import jax
import jax.numpy as jnp
from jax import lax
from jax.experimental import pallas as pl
from jax.experimental.pallas import tpu as pltpu
from jax.experimental.pallas import tpu_sc as plsc

F32 = jnp.float32

_N = 10000
_E = 320000
_DF = 128
_DE = 16
_DG = 16
_L = 16

_NC = 2
_NS = 16
_NW = _NC * _NS
_EPW = _E // _NW
_TB = 80
_CH = 2000
_TPC = _CH // _TB
_NCHUNK = _EPW // _CH
_NPS = _N // _NS

def _sc_gather_body(ps_hbm, pr_hbm, snd_hbm, rcv_hbm, gs_hbm, gr_hbm,
                    idx_s, idx_r, rows_s, rows_r, sem):
    cid = lax.axis_index("c")
    sid = lax.axis_index("s")
    wid = cid * _NS + sid
    base_e = wid * _EPW

    pltpu.sync_copy(snd_hbm.at[pl.ds(base_e, _EPW)], idx_s)
    pltpu.sync_copy(rcv_hbm.at[pl.ds(base_e, _EPW)], idx_r)

    def chunk(t, carry):
        descs = []
        for j in range(_TPC):
            src = pl.ds((t * _TPC + j) * _TB, _TB)
            dst = pl.ds(j * _TB, _TB)
            descs.append(pltpu.async_copy(ps_hbm.at[idx_s.at[src]],
                                          rows_s.at[dst], sem))
            descs.append(pltpu.async_copy(pr_hbm.at[idx_r.at[src]],
                                          rows_r.at[dst], sem))
        for d in descs:
            d.wait()
        pltpu.sync_copy(rows_s, gs_hbm.at[pl.ds(base_e + t * _CH, _CH)])
        pltpu.sync_copy(rows_r, gr_hbm.at[pl.ds(base_e + t * _CH, _CH)])
        return carry

    lax.fori_loop(0, _NCHUNK, chunk, 0)


def _sc_gather(ps, pr, snd, rcv):
    mesh = plsc.VectorSubcoreMesh(core_axis_name="c", subcore_axis_name="s")
    k = functools.partial(
        pl.kernel,
        out_type=(
            jax.ShapeDtypeStruct((_E, _L), F32),
            jax.ShapeDtypeStruct((_E, _L), F32),
        ),
        mesh=mesh,
        scratch_types=[
            pltpu.VMEM((_EPW,), jnp.int32),
            pltpu.VMEM((_EPW,), jnp.int32),
            pltpu.VMEM((_CH, _L), F32),
            pltpu.VMEM((_CH, _L), F32),
            pltpu.SemaphoreType.DMA,
        ],
        compiler_params=pltpu.CompilerParams(use_tc_tiling_on_sc=False),
    )(_sc_gather_body)
    return k(ps, pr, snd, rcv)


def _sc_scatter_body(ne_hbm, snd_hbm, rcv_hbm, out_hbm,
                     idx_s, idx_r, rows, zbuf, sh_s, sh_r):
    cid = lax.axis_index("c")
    sid = lax.axis_index("s")
    wid = cid * _NS + sid
    zslice = pl.ds(sid * _NPS, _NPS)

    def zrow(i, c):
        zbuf[i, :] = jnp.zeros((_L,), F32)
        return c

    lax.fori_loop(0, _NPS, zrow, 0)
    pltpu.sync_copy(zbuf, sh_s.at[zslice])
    pltpu.sync_copy(zbuf, sh_r.at[zslice])
    pltpu.sync_copy(snd_hbm.at[pl.ds(wid * _EPW, _EPW)], idx_s)
    pltpu.sync_copy(rcv_hbm.at[pl.ds(wid * _EPW, _EPW)], idx_r)
    plsc.subcore_barrier()

    def chunk(t, carry):
        pltpu.sync_copy(ne_hbm.at[pl.ds(wid * _EPW + t * _CH, _CH)], rows)
        for j in range(_TPC):
            isl = pl.ds((t * _TPC + j) * _TB, _TB)
            src = rows.at[pl.ds(j * _TB, _TB)]
            pltpu.sync_copy(src, sh_s.at[idx_s.at[isl]], add=True)
            pltpu.sync_copy(src, sh_r.at[idx_r.at[isl]], add=True)
        return carry

    lax.fori_loop(0, _NCHUNK, chunk, 0)
    plsc.subcore_barrier()

    pltpu.sync_copy(sh_s.at[zslice], out_hbm.at[cid, 0, zslice])
    pltpu.sync_copy(sh_r.at[zslice], out_hbm.at[cid, 1, zslice])


def _sc_scatter(ne, snd, rcv):
    mesh = plsc.VectorSubcoreMesh(core_axis_name="c", subcore_axis_name="s")
    k = functools.partial(
        pl.kernel,
        out_type=jax.ShapeDtypeStruct((_NC, 2, _N, _L), F32),
        mesh=mesh,
        scratch_types=[
            pltpu.VMEM((_EPW,), jnp.int32),
            pltpu.VMEM((_EPW,), jnp.int32),
            pltpu.VMEM((_CH, _L), F32),
            pltpu.VMEM((_NPS, _L), F32),
            pltpu.VMEM_SHARED((_N, _L), F32),
            pltpu.VMEM_SHARED((_N, _L), F32),
        ],
        compiler_params=pltpu.CompilerParams(use_tc_tiling_on_sc=False),
    )(_sc_scatter_body)
    return k(ne, snd, rcv)


def _tc_project_body(n_ref, ws_ref, wr_ref, ps_ref, pr_ref):
    n = n_ref[...]
    ps_ref[...] = jnp.dot(n, ws_ref[...], preferred_element_type=F32)
    pr_ref[...] = jnp.dot(n, wr_ref[...], preferred_element_type=F32)


def _tc_project(nodes, ws, wr):
    bn = 2000
    grid = _N // bn
    return pl.pallas_call(
        _tc_project_body,
        grid=(grid,),
        in_specs=[
            pl.BlockSpec((bn, _DF), lambda i: (i, 0)),
            pl.BlockSpec((_DF, _L), lambda i: (0, 0)),
            pl.BlockSpec((_DF, _L), lambda i: (0, 0)),
        ],
        out_specs=[
            pl.BlockSpec((bn, _L), lambda i: (i, 0)),
            pl.BlockSpec((bn, _L), lambda i: (i, 0)),
        ],
        out_shape=[
            jax.ShapeDtypeStruct((_N, _L), F32),
            jax.ShapeDtypeStruct((_N, _L), F32),
        ],
    )(nodes, ws, wr)


_EP = _E // 8


def _tc_edge_body(e_ref, gs_ref, gr_ref, web1_ref, we1gt_ref, be1t_ref,
                  web2_ref, be2t_ref, g_ref, ne_ref, esum_ref):
    cst = (jnp.dot(g_ref[...], we1gt_ref[...], preferred_element_type=F32)
           + be1t_ref[...])
    pre = (jnp.dot(e_ref[...], web1_ref[...], preferred_element_type=F32)
           + gs_ref[...] + gr_ref[...] + cst)
    h = jnp.maximum(pre, 0.0)
    ne = jnp.dot(h, web2_ref[...], preferred_element_type=F32) + be2t_ref[...]
    ne_ref[...] = ne
    esum_ref[0, 0, :] = jnp.sum(ne, axis=0)


def _tc_edge(edges_p, gs_p, gr_p, web1, we1gt, be1t, web2, be2t, g):
    be = 4000
    grid = _EP // be
    return pl.pallas_call(
        _tc_edge_body,
        grid=(grid,),
        in_specs=[
            pl.BlockSpec((be, 128), lambda i: (i, 0)),
            pl.BlockSpec((be, 128), lambda i: (i, 0)),
            pl.BlockSpec((be, 128), lambda i: (i, 0)),
            pl.BlockSpec((128, 128), lambda i: (0, 0)),
            pl.BlockSpec((_DG, 128), lambda i: (0, 0)),
            pl.BlockSpec((1, 128), lambda i: (0, 0)),
            pl.BlockSpec((128, 128), lambda i: (0, 0)),
            pl.BlockSpec((1, 128), lambda i: (0, 0)),
            pl.BlockSpec((1, _DG), lambda i: (0, 0)),
        ],
        out_specs=[
            pl.BlockSpec((be, 128), lambda i: (i, 0)),
            pl.BlockSpec((1, 1, 128), lambda i: (i, 0, 0)),
        ],
        out_shape=[
            jax.ShapeDtypeStruct((_EP, 128), F32),
            jax.ShapeDtypeStruct((grid, 1, 128), F32),
        ],
    )(edges_p, gs_p, gr_p, web1, we1gt, be1t, web2, be2t, g)


def _tc_node_body(n_ref, p00_ref, p01_ref, p10_ref, p11_ref, esum_ref,
                  wn1x_ref, wn1s_ref, wn1r_ref, wn1g_ref, bn1_ref,
                  wn2_ref, bn2_ref, g_ref,
                  wg1n_ref, wg1e_ref, wg1g_ref, bg1_ref, wg2_ref, bg2_ref,
                  nn_ref, ng_ref, nsum_ref):
    sagg = p00_ref[0, 0] + p10_ref[0, 0]
    ragg = p01_ref[0, 0] + p11_ref[0, 0]
    cst = (jnp.dot(g_ref[...], wn1g_ref[...], preferred_element_type=F32)
           + bn1_ref[...])
    pre = (jnp.dot(n_ref[...], wn1x_ref[...], preferred_element_type=F32)
           + jnp.dot(sagg, wn1s_ref[...], preferred_element_type=F32)
           + jnp.dot(ragg, wn1r_ref[...], preferred_element_type=F32)
           + cst)
    nn = (jnp.dot(jnp.maximum(pre, 0.0), wn2_ref[...],
                  preferred_element_type=F32) + bn2_ref[...])
    nn_ref[...] = nn

    i = pl.program_id(0)
    s = jnp.sum(nn, axis=0, keepdims=True)

    @pl.when(i == 0)
    def _():
        nsum_ref[...] = s

    @pl.when(i > 0)
    def _():
        nsum_ref[...] = nsum_ref[...] + s

    @pl.when(i == pl.num_programs(0) - 1)
    def _():
        node_mean = nsum_ref[...] * (1.0 / _N)
        etot = jnp.sum(esum_ref[...], axis=(0, 1)).reshape(1, 128)
        gpre = (jnp.dot(node_mean, wg1n_ref[...], preferred_element_type=F32)
                + jnp.dot(etot, wg1e_ref[...],
                          preferred_element_type=F32) * (1.0 / _E)
                + jnp.dot(g_ref[...], wg1g_ref[...], preferred_element_type=F32)
                + bg1_ref[...])
        ng_ref[...] = (jnp.dot(jnp.maximum(gpre, 0.0), wg2_ref[...],
                               preferred_element_type=F32) + bg2_ref[...])


def _tc_node(nodes, partials, esums,
             wn1x, wn1s, wn1r, wn1g, bn1, wn2, bn2, g,
             wg1n, wg1e, wg1g, bg1, wg2, bg2):
    bn = 2000
    grid = _N // bn
    nblk = esums.shape[0]
    w16 = lambda i: (0, 0)
    return pl.pallas_call(
        _tc_node_body,
        grid=(grid,),
        in_specs=[
            pl.BlockSpec((bn, _DF), lambda i: (i, 0)),
            pl.BlockSpec((1, 1, bn, _L), lambda i: (0, 0, i, 0)),
            pl.BlockSpec((1, 1, bn, _L), lambda i: (0, 1, i, 0)),
            pl.BlockSpec((1, 1, bn, _L), lambda i: (1, 0, i, 0)),
            pl.BlockSpec((1, 1, bn, _L), lambda i: (1, 1, i, 0)),
            pl.BlockSpec((nblk, 1, 128), lambda i: (0, 0, 0)),
            pl.BlockSpec((_DF, _L), w16),
            pl.BlockSpec((_L, _L), w16),
            pl.BlockSpec((_L, _L), w16),
            pl.BlockSpec((_DG, _L), w16),
            pl.BlockSpec((1, _L), w16),
            pl.BlockSpec((_L, _L), w16),
            pl.BlockSpec((1, _L), w16),
            pl.BlockSpec((1, _DG), w16),
            pl.BlockSpec((_L, _L), w16),
            pl.BlockSpec((128, _L), w16),
            pl.BlockSpec((_DG, _L), w16),
            pl.BlockSpec((1, _L), w16),
            pl.BlockSpec((_L, _L), w16),
            pl.BlockSpec((1, _L), w16),
        ],
        out_specs=[
            pl.BlockSpec((bn, _L), lambda i: (i, 0)),
            pl.BlockSpec((1, _L), lambda i: (0, 0)),
        ],
        out_shape=[
            jax.ShapeDtypeStruct((_N, _L), F32),
            jax.ShapeDtypeStruct((1, _L), F32),
        ],
        scratch_shapes=[pltpu.VMEM((1, _L), F32)],
    )(nodes, partials, partials, partials, partials, esums,
      wn1x, wn1s, wn1r, wn1g, bn1, wn2, bn2, g,
      wg1n, wg1e, wg1g, bg1, wg2, bg2)


def kernel(nodes, edges, senders, receivers, globals_,
           We1, be1, We2, be2, Wn1, bn1, Wn2, bn2, Wg1, bg1, Wg2, bg2):
    we1e = We1[:_DE]
    we1s = We1[_DE:_DE + _DF]
    we1r = We1[_DE + _DF:_DE + 2 * _DF]
    we1g = We1[_DE + 2 * _DF:]
    wn1x = Wn1[:_DF]
    wn1s = Wn1[_DF:_DF + _L]
    wn1r = Wn1[_DF + _L:_DF + 2 * _L]
    wn1g = Wn1[_DF + 2 * _L:]
    wg1n = Wg1[:_L]
    wg1e = Wg1[_L:2 * _L]
    wg1g = Wg1[2 * _L:]
    be1r = be1.reshape(1, _L)
    be2r = be2.reshape(1, _L)
    bn1r = bn1.reshape(1, _L)
    bn2r = bn2.reshape(1, _L)
    bg1r = bg1.reshape(1, _L)
    bg2r = bg2.reshape(1, _L)
    eye8 = jnp.eye(8, dtype=F32)
    web1 = jnp.kron(eye8, we1e)
    web2 = jnp.kron(eye8, We2)
    we1gt = jnp.tile(we1g, (1, 8))
    be1t = jnp.tile(be1r, (1, 8))
    be2t = jnp.tile(be2r, (1, 8))
    wg1et = jnp.tile(wg1e, (8, 1))

    edges_p = edges.reshape(_EP, 128)

    ps, pr = _tc_project(nodes, we1s, we1r)
    gs, gr = _sc_gather(ps, pr, senders, receivers)
    ne_p, esums = _tc_edge(edges_p, gs.reshape(_EP, 128), gr.reshape(_EP, 128),
                           web1, we1gt, be1t, web2, be2t, globals_)
    new_edges = ne_p.reshape(_E, _L)
    partials = _sc_scatter(new_edges, senders, receivers)
    new_nodes, new_globals = _tc_node(
        nodes, partials, esums, wn1x, wn1s, wn1r, wn1g, bn1r, Wn2, bn2r,
        globals_, wg1n, wg1et, wg1g, bg1r, Wg2, bg2r)
    return new_nodes, new_edges, new_globals


# speedup vs baseline: 9.4833x; 1.0263x over previous
import functools

import jax
import jax.numpy as jnp
from jax import lax
from jax.experimental import pallas as pl
from jax.experimental.pallas import tpu as pltpu
from jax.experimental.pallas import tpu_sc as plsc

F32 = jnp.float32
_PREC = jax.lax.Precision.DEFAULT

_N = 10000
_E = 320000
_DF = 128
_DE = 16
_DG = 16
_L = 16

_NC = 2
_NS = 16
_NW = _NC * _NS
_EPW = _E // _NW
_TB = 80
_CH = 2000
_TPC = _CH // _TB
_NCHUNK = _EPW // _CH
_NPS = _N // _NS

def _sc_gather_body(ps_hbm, pr_hbm, snd_hbm, rcv_hbm, gs_hbm, gr_hbm,
                    idx_s, idx_r, rows_s, rows_r, sem):
    cid = lax.axis_index("c")
    sid = lax.axis_index("s")
    wid = cid * _NS + sid
    base_e = wid * _EPW

    pltpu.sync_copy(snd_hbm.at[pl.ds(base_e, _EPW)], idx_s)
    pltpu.sync_copy(rcv_hbm.at[pl.ds(base_e, _EPW)], idx_r)

    def chunk(t, carry):
        descs = []
        for j in range(_TPC):
            src = pl.ds((t * _TPC + j) * _TB, _TB)
            dst = pl.ds(j * _TB, _TB)
            descs.append(pltpu.async_copy(ps_hbm.at[idx_s.at[src]],
                                          rows_s.at[dst], sem))
            descs.append(pltpu.async_copy(pr_hbm.at[idx_r.at[src]],
                                          rows_r.at[dst], sem))
        for d in descs:
            d.wait()
        pltpu.sync_copy(rows_s, gs_hbm.at[pl.ds(base_e + t * _CH, _CH)])
        pltpu.sync_copy(rows_r, gr_hbm.at[pl.ds(base_e + t * _CH, _CH)])
        return carry

    lax.fori_loop(0, _NCHUNK, chunk, 0)


def _sc_gather(ps, pr, snd, rcv):
    mesh = plsc.VectorSubcoreMesh(core_axis_name="c", subcore_axis_name="s")
    k = functools.partial(
        pl.kernel,
        out_type=(
            jax.ShapeDtypeStruct((_E, _L), F32),
            jax.ShapeDtypeStruct((_E, _L), F32),
        ),
        mesh=mesh,
        scratch_types=[
            pltpu.VMEM((_EPW,), jnp.int32),
            pltpu.VMEM((_EPW,), jnp.int32),
            pltpu.VMEM((_CH, _L), F32),
            pltpu.VMEM((_CH, _L), F32),
            pltpu.SemaphoreType.DMA,
        ],
        compiler_params=pltpu.CompilerParams(use_tc_tiling_on_sc=False),
    )(_sc_gather_body)
    return k(ps, pr, snd, rcv)


def _sc_scatter_body(ne_hbm, snd_hbm, rcv_hbm, out_hbm,
                     idx_s, idx_r, rows, zbuf, sh_s, sh_r):
    cid = lax.axis_index("c")
    sid = lax.axis_index("s")
    wid = cid * _NS + sid
    zslice = pl.ds(sid * _NPS, _NPS)

    def zrow(i, c):
        zbuf[i, :] = jnp.zeros((_L,), F32)
        return c

    lax.fori_loop(0, _NPS, zrow, 0)
    pltpu.sync_copy(zbuf, sh_s.at[zslice])
    pltpu.sync_copy(zbuf, sh_r.at[zslice])
    pltpu.sync_copy(snd_hbm.at[pl.ds(wid * _EPW, _EPW)], idx_s)
    pltpu.sync_copy(rcv_hbm.at[pl.ds(wid * _EPW, _EPW)], idx_r)
    plsc.subcore_barrier()

    def chunk(t, carry):
        pltpu.sync_copy(ne_hbm.at[pl.ds(wid * _EPW + t * _CH, _CH)], rows)
        for j in range(_TPC):
            isl = pl.ds((t * _TPC + j) * _TB, _TB)
            src = rows.at[pl.ds(j * _TB, _TB)]
            pltpu.sync_copy(src, sh_s.at[idx_s.at[isl]], add=True)
            pltpu.sync_copy(src, sh_r.at[idx_r.at[isl]], add=True)
        return carry

    lax.fori_loop(0, _NCHUNK, chunk, 0)
    plsc.subcore_barrier()

    pltpu.sync_copy(sh_s.at[zslice], out_hbm.at[cid, 0, zslice])
    pltpu.sync_copy(sh_r.at[zslice], out_hbm.at[cid, 1, zslice])


def _sc_scatter(ne, snd, rcv):
    mesh = plsc.VectorSubcoreMesh(core_axis_name="c", subcore_axis_name="s")
    k = functools.partial(
        pl.kernel,
        out_type=jax.ShapeDtypeStruct((_NC, 2, _N, _L), F32),
        mesh=mesh,
        scratch_types=[
            pltpu.VMEM((_EPW,), jnp.int32),
            pltpu.VMEM((_EPW,), jnp.int32),
            pltpu.VMEM((_CH, _L), F32),
            pltpu.VMEM((_NPS, _L), F32),
            pltpu.VMEM_SHARED((_N, _L), F32),
            pltpu.VMEM_SHARED((_N, _L), F32),
        ],
        compiler_params=pltpu.CompilerParams(use_tc_tiling_on_sc=False),
    )(_sc_scatter_body)
    return k(ne, snd, rcv)


_NP8 = _N // 8


def _dot8(n, w_ref):
    acc = None
    for j in range(8):
        p = jnp.dot(n[:, 128 * j:128 * (j + 1)],
                    w_ref[128 * j:128 * (j + 1), :],
                    preferred_element_type=F32, precision=_PREC)
        acc = p if acc is None else acc + p
    return acc


def _tc_project_body(n_ref, ws_ref, wr_ref, ps_ref, pr_ref):
    n = n_ref[...]
    ps_ref[...] = _dot8(n, ws_ref)
    pr_ref[...] = _dot8(n, wr_ref)


def _tc_project(nodes_r, wsb, wrb):
    bn = _NP8
    grid = _NP8 // bn
    return pl.pallas_call(
        _tc_project_body,
        grid=(grid,),
        in_specs=[
            pl.BlockSpec((bn, 8 * _DF), lambda i: (i, 0)),
            pl.BlockSpec((8 * _DF, 128), lambda i: (0, 0)),
            pl.BlockSpec((8 * _DF, 128), lambda i: (0, 0)),
        ],
        out_specs=[
            pl.BlockSpec((bn, 128), lambda i: (i, 0)),
            pl.BlockSpec((bn, 128), lambda i: (i, 0)),
        ],
        out_shape=[
            jax.ShapeDtypeStruct((_NP8, 128), F32),
            jax.ShapeDtypeStruct((_NP8, 128), F32),
        ],
    )(nodes_r, wsb, wrb)


_EP = _E // 8


def _tc_edge_body(e_ref, gs_ref, gr_ref, web1_ref, we1gt_ref, be1t_ref,
                  web2_ref, be2t_ref, g_ref, ne_ref, esum_ref):
    cst = (jnp.dot(g_ref[...], we1gt_ref[...], preferred_element_type=F32, precision=_PREC)
           + be1t_ref[...])
    pre = (jnp.dot(e_ref[...], web1_ref[...], preferred_element_type=F32, precision=_PREC)
           + gs_ref[...] + gr_ref[...] + cst)
    h = jnp.maximum(pre, 0.0)
    ne = jnp.dot(h, web2_ref[...], preferred_element_type=F32, precision=_PREC) + be2t_ref[...]
    ne_ref[...] = ne
    esum_ref[0, 0, :] = jnp.sum(ne, axis=0)


def _tc_edge(edges_p, gs_p, gr_p, web1, we1gt, be1t, web2, be2t, g):
    be = 4000
    grid = _EP // be
    return pl.pallas_call(
        _tc_edge_body,
        grid=(grid,),
        in_specs=[
            pl.BlockSpec((be, 128), lambda i: (i, 0)),
            pl.BlockSpec((be, 128), lambda i: (i, 0)),
            pl.BlockSpec((be, 128), lambda i: (i, 0)),
            pl.BlockSpec((128, 128), lambda i: (0, 0)),
            pl.BlockSpec((_DG, 128), lambda i: (0, 0)),
            pl.BlockSpec((1, 128), lambda i: (0, 0)),
            pl.BlockSpec((128, 128), lambda i: (0, 0)),
            pl.BlockSpec((1, 128), lambda i: (0, 0)),
            pl.BlockSpec((1, _DG), lambda i: (0, 0)),
        ],
        out_specs=[
            pl.BlockSpec((be, 128), lambda i: (i, 0)),
            pl.BlockSpec((1, 1, 128), lambda i: (i, 0, 0)),
        ],
        out_shape=[
            jax.ShapeDtypeStruct((_EP, 128), F32),
            jax.ShapeDtypeStruct((grid, 1, 128), F32),
        ],
    )(edges_p, gs_p, gr_p, web1, we1gt, be1t, web2, be2t, g)


def _tc_node_body(n_ref, p00_ref, p01_ref, p10_ref, p11_ref, esum_ref,
                  wn1x_ref, wn1s_ref, wn1r_ref, wn1g_ref, bn1_ref,
                  wn2_ref, bn2_ref, g_ref, sel_ref,
                  wg1n_ref, wg1e_ref, wg1g_ref, bg1_ref, wg2_ref, bg2_ref,
                  nn_ref, ng_ref, nsum_ref):
    sagg = p00_ref[0, 0] + p10_ref[0, 0]
    ragg = p01_ref[0, 0] + p11_ref[0, 0]
    cst = (jnp.dot(g_ref[...], wn1g_ref[...], preferred_element_type=F32, precision=_PREC)
           + bn1_ref[...])
    pre = (_dot8(n_ref[...], wn1x_ref)
           + jnp.dot(sagg, wn1s_ref[...], preferred_element_type=F32, precision=_PREC)
           + jnp.dot(ragg, wn1r_ref[...], preferred_element_type=F32, precision=_PREC)
           + cst)
    nn = (jnp.dot(jnp.maximum(pre, 0.0), wn2_ref[...],
                  preferred_element_type=F32, precision=_PREC) + bn2_ref[...])
    nn_ref[...] = nn

    i = pl.program_id(0)
    s = jnp.sum(nn, axis=0, keepdims=True)

    @pl.when(i == 0)
    def _():
        nsum_ref[...] = s

    @pl.when(i > 0)
    def _():
        nsum_ref[...] = nsum_ref[...] + s

    @pl.when(i == pl.num_programs(0) - 1)
    def _():
        nsum16 = jnp.dot(nsum_ref[...], sel_ref[...],
                         preferred_element_type=F32,
                         precision=jax.lax.Precision.HIGHEST)
        esum16 = jnp.dot(jnp.sum(esum_ref[...], axis=(0, 1)).reshape(1, 128),
                         sel_ref[...], preferred_element_type=F32,
                         precision=jax.lax.Precision.HIGHEST)
        gpre = (jnp.dot(nsum16 * (1.0 / _N), wg1n_ref[...],
                        preferred_element_type=F32, precision=_PREC)
                + jnp.dot(esum16 * (1.0 / _E), wg1e_ref[...],
                          preferred_element_type=F32, precision=_PREC)
                + jnp.dot(g_ref[...], wg1g_ref[...], preferred_element_type=F32, precision=_PREC)
                + bg1_ref[...])
        ng_ref[...] = (jnp.dot(jnp.maximum(gpre, 0.0), wg2_ref[...],
                               preferred_element_type=F32, precision=_PREC) + bg2_ref[...])


def _tc_node(nodes_r, partials_p, esums,
             wn1xb, wn1sb, wn1rb, wn1gt, bn1t, wn2b, bn2t, g, sel,
             wg1n, wg1e, wg1g, bg1, wg2, bg2):
    bn = _NP8
    grid = _NP8 // bn
    nblk = esums.shape[0]
    w16 = lambda i: (0, 0)
    return pl.pallas_call(
        _tc_node_body,
        grid=(grid,),
        in_specs=[
            pl.BlockSpec((bn, 8 * _DF), lambda i: (i, 0)),
            pl.BlockSpec((1, 1, bn, 128), lambda i: (0, 0, i, 0)),
            pl.BlockSpec((1, 1, bn, 128), lambda i: (0, 1, i, 0)),
            pl.BlockSpec((1, 1, bn, 128), lambda i: (1, 0, i, 0)),
            pl.BlockSpec((1, 1, bn, 128), lambda i: (1, 1, i, 0)),
            pl.BlockSpec((nblk, 1, 128), lambda i: (0, 0, 0)),
            pl.BlockSpec((8 * _DF, 128), w16),
            pl.BlockSpec((128, 128), w16),
            pl.BlockSpec((128, 128), w16),
            pl.BlockSpec((_DG, 128), w16),
            pl.BlockSpec((1, 128), w16),
            pl.BlockSpec((128, 128), w16),
            pl.BlockSpec((1, 128), w16),
            pl.BlockSpec((1, _DG), w16),
            pl.BlockSpec((128, _L), w16),
            pl.BlockSpec((_L, _L), w16),
            pl.BlockSpec((_L, _L), w16),
            pl.BlockSpec((_DG, _L), w16),
            pl.BlockSpec((1, _L), w16),
            pl.BlockSpec((_L, _L), w16),
            pl.BlockSpec((1, _L), w16),
        ],
        out_specs=[
            pl.BlockSpec((bn, 128), lambda i: (i, 0)),
            pl.BlockSpec((1, _L), lambda i: (0, 0)),
        ],
        out_shape=[
            jax.ShapeDtypeStruct((_NP8, 128), F32),
            jax.ShapeDtypeStruct((1, _L), F32),
        ],
        scratch_shapes=[pltpu.VMEM((1, 128), F32)],
    )(nodes_r, partials_p, partials_p, partials_p, partials_p, esums,
      wn1xb, wn1sb, wn1rb, wn1gt, bn1t, wn2b, bn2t, g, sel,
      wg1n, wg1e, wg1g, bg1, wg2, bg2)


def kernel(nodes, edges, senders, receivers, globals_,
           We1, be1, We2, be2, Wn1, bn1, Wn2, bn2, Wg1, bg1, Wg2, bg2):
    we1e = We1[:_DE]
    we1s = We1[_DE:_DE + _DF]
    we1r = We1[_DE + _DF:_DE + 2 * _DF]
    we1g = We1[_DE + 2 * _DF:]
    wn1x = Wn1[:_DF]
    wn1s = Wn1[_DF:_DF + _L]
    wn1r = Wn1[_DF + _L:_DF + 2 * _L]
    wn1g = Wn1[_DF + 2 * _L:]
    wg1n = Wg1[:_L]
    wg1e = Wg1[_L:2 * _L]
    wg1g = Wg1[2 * _L:]
    be1r = be1.reshape(1, _L)
    be2r = be2.reshape(1, _L)
    bn1r = bn1.reshape(1, _L)
    bn2r = bn2.reshape(1, _L)
    bg1r = bg1.reshape(1, _L)
    bg2r = bg2.reshape(1, _L)
    eye8 = jnp.eye(8, dtype=F32)
    web1 = jnp.kron(eye8, we1e)
    web2 = jnp.kron(eye8, We2)
    we1gt = jnp.tile(we1g, (1, 8))
    be1t = jnp.tile(be1r, (1, 8))
    be2t = jnp.tile(be2r, (1, 8))
    wsb = jnp.kron(eye8, we1s)
    wrb = jnp.kron(eye8, we1r)
    wn1xb = jnp.kron(eye8, wn1x)
    wn1sb = jnp.kron(eye8, wn1s)
    wn1rb = jnp.kron(eye8, wn1r)
    wn2b = jnp.kron(eye8, Wn2)
    wn1gt = jnp.tile(wn1g, (1, 8))
    bn1t = jnp.tile(bn1r, (1, 8))
    bn2t = jnp.tile(bn2r, (1, 8))
    sel16 = jnp.tile(jnp.eye(_L, dtype=F32), (8, 1))

    edges_p = edges.reshape(_EP, 128)
    nodes_r = nodes.reshape(_NP8, 8 * _DF)

    ps_p, pr_p = _tc_project(nodes_r, wsb, wrb)
    gs, gr = _sc_gather(ps_p.reshape(_N, _L), pr_p.reshape(_N, _L),
                        senders, receivers)
    ne_p, esums = _tc_edge(edges_p, gs.reshape(_EP, 128), gr.reshape(_EP, 128),
                           web1, we1gt, be1t, web2, be2t, globals_)
    new_edges = ne_p.reshape(_E, _L)
    partials = _sc_scatter(new_edges, senders, receivers)
    nn_p, new_globals = _tc_node(
        nodes_r, partials.reshape(_NC, 2, _NP8, 128), esums,
        wn1xb, wn1sb, wn1rb, wn1gt, bn1t, wn2b, bn2t,
        globals_, sel16, wg1n, wg1e, wg1g, bg1r, Wg2, bg2r)
    return nn_p.reshape(_N, _L), new_edges, new_globals
